# baseline (speedup 1.0000x reference)
import jax
import jax.numpy as jnp
from jax.experimental import pallas as pl

BATCH = 32
IN_DIM = 512
EMBED_DIM = 32
NUM_VARS = 1024
TOP_K = 32

ROWS = 512
N_SEARCH = 18


def _fused_kernel(cvr_ref, cvc_ref, w1t_ref, b1r_ref, w2t_ref, b2r_ref,
                  wgt_ref, bg_ref, w1_ref, b1c_ref, w2_ref, b2c_ref,
                  vrows_ref, vt_ref, vfull_ref, out_ref):
    h_r = jax.nn.relu(
        jnp.dot(cvr_ref[0], w1t_ref[...], preferred_element_type=jnp.float32)
        + b1r_ref[...])
    c_r = jnp.dot(h_r, w2t_ref[...], preferred_element_type=jnp.float32) + b2r_ref[...]
    gate = jax.nn.sigmoid(
        jnp.dot(c_r, wgt_ref[...], preferred_element_type=jnp.float32)
        + bg_ref[...])[0, 0]

    h_c = jax.nn.relu(
        jnp.dot(w1_ref[...], cvc_ref[0], preferred_element_type=jnp.float32)
        + b1c_ref[...])
    c_c = jnp.dot(w2_ref[...], h_c, preferred_element_type=jnp.float32) + b2c_ref[...]

    sim_rows = vrows_ref[...] * c_r
    sim_t = vt_ref[...] * c_c
    logits = jnp.dot(sim_rows, sim_t, preferred_element_type=jnp.float32)

    inv_n = 1.0 / NUM_VARS
    s_sum = jnp.sum(sim_t, axis=1, keepdims=True)
    mu = jnp.dot(sim_rows, s_sum, preferred_element_type=jnp.float32) * inv_n
    gram = jnp.dot(vt_ref[...], vfull_ref[...],
                   preferred_element_type=jnp.float32)
    m_mat = (c_c * c_r) * gram
    a_rows = jnp.dot(sim_rows, m_mat, preferred_element_type=jnp.float32)
    m2 = jnp.sum(a_rows * sim_rows, axis=1, keepdims=True) * inv_n
    sd = jnp.sqrt(jnp.maximum(m2 - mu * mu, 1e-12))
    lo = mu - 4.0 * sd
    hi = mu + 8.0 * sd
    ge = None
    for it in range(N_SEARCH):
        if it == 0:
            mid = mu + 1.8627 * sd
        elif it == 1:
            mid = jnp.where(ge, mu + 2.35 * sd, mu + 1.45 * sd)
        else:
            mid = 0.5 * (lo + hi)
        cnt = jnp.sum((logits >= mid).astype(jnp.float32), axis=1, keepdims=True)
        ge = cnt >= TOP_K
        lo = jnp.where(ge, mid, lo)
        hi = jnp.where(ge, hi, mid)

    sig = 0.5 + 0.5 * jnp.tanh(0.5 * logits)
    out = jnp.where(logits >= lo, sig * gate, 0.0)
    out_ref[...] = out[None]


@jax.jit
def kernel(context_vec, var_emb, W1, b1, W2, b2, Wg, bg):
    grid = (BATCH, NUM_VARS // ROWS)
    out = pl.pallas_call(
        _fused_kernel,
        grid=grid,
        in_specs=[
            pl.BlockSpec((1, 1, IN_DIM), lambda b, r: (b, 0, 0)),
            pl.BlockSpec((1, IN_DIM, 1), lambda b, r: (b, 0, 0)),
            pl.BlockSpec((IN_DIM, EMBED_DIM), lambda b, r: (0, 0)),
            pl.BlockSpec((1, EMBED_DIM), lambda b, r: (0, 0)),
            pl.BlockSpec((EMBED_DIM, EMBED_DIM), lambda b, r: (0, 0)),
            pl.BlockSpec((1, EMBED_DIM), lambda b, r: (0, 0)),
            pl.BlockSpec((EMBED_DIM, 1), lambda b, r: (0, 0)),
            pl.BlockSpec((1, 1), lambda b, r: (0, 0)),
            pl.BlockSpec((EMBED_DIM, IN_DIM), lambda b, r: (0, 0)),
            pl.BlockSpec((EMBED_DIM, 1), lambda b, r: (0, 0)),
            pl.BlockSpec((EMBED_DIM, EMBED_DIM), lambda b, r: (0, 0)),
            pl.BlockSpec((EMBED_DIM, 1), lambda b, r: (0, 0)),
            pl.BlockSpec((ROWS, EMBED_DIM), lambda b, r: (r, 0)),
            pl.BlockSpec((EMBED_DIM, NUM_VARS), lambda b, r: (0, 0)),
            pl.BlockSpec((NUM_VARS, EMBED_DIM), lambda b, r: (0, 0)),
        ],
        out_specs=pl.BlockSpec((1, ROWS, NUM_VARS), lambda b, r: (b, r, 0)),
        out_shape=jax.ShapeDtypeStruct((BATCH, NUM_VARS, NUM_VARS), jnp.float32),
    )(
        context_vec.reshape(BATCH, 1, IN_DIM),
        context_vec.reshape(BATCH, IN_DIM, 1),
        W1.T,
        b1.reshape(1, EMBED_DIM),
        W2.T,
        b2.reshape(1, EMBED_DIM),
        Wg.T,
        bg.reshape(1, 1),
        W1,
        b1.reshape(EMBED_DIM, 1),
        W2,
        b2.reshape(EMBED_DIM, 1),
        var_emb,
        var_emb.T,
        var_emb,
    )
    return out


# speedup vs baseline: 1.1986x; 1.1986x over previous
import jax
import jax.numpy as jnp
from jax.experimental import pallas as pl

BATCH = 32
IN_DIM = 512
EMBED_DIM = 32
NUM_VARS = 1024
TOP_K = 32

ROWS = 1024
N_SEARCH = 18


def _fused_kernel(cvr_ref, cvc_ref, w1t_ref, b1r_ref, w2t_ref, b2r_ref,
                  wgt_ref, bg_ref, w1_ref, b1c_ref, w2_ref, b2c_ref,
                  vrows_ref, vt_ref, out_ref):
    h_r = jax.nn.relu(
        jnp.dot(cvr_ref[0], w1t_ref[...], preferred_element_type=jnp.float32)
        + b1r_ref[...])
    c_r = jnp.dot(h_r, w2t_ref[...], preferred_element_type=jnp.float32) + b2r_ref[...]
    gate = jax.nn.sigmoid(
        jnp.dot(c_r, wgt_ref[...], preferred_element_type=jnp.float32)
        + bg_ref[...])[0, 0]

    h_c = jax.nn.relu(
        jnp.dot(w1_ref[...], cvc_ref[0], preferred_element_type=jnp.float32)
        + b1c_ref[...])
    c_c = jnp.dot(w2_ref[...], h_c, preferred_element_type=jnp.float32) + b2c_ref[...]

    sim_rows = vrows_ref[...] * c_r
    sim_t = vt_ref[...] * c_c
    logits = jnp.dot(sim_rows, sim_t, preferred_element_type=jnp.float32)

    mu = jnp.mean(logits, axis=1, keepdims=True)
    m2 = jnp.mean(logits * logits, axis=1, keepdims=True)
    sd = jnp.sqrt(jnp.maximum(m2 - mu * mu, 1e-12))
    lo = mu - 4.0 * sd
    hi = mu + 6.0 * sd
    ge = None
    for it in range(N_SEARCH):
        if it == 0:
            mid = mu + 1.8627 * sd
        elif it == 1:
            mid = jnp.where(ge, mu + 2.35 * sd, mu + 1.45 * sd)
        else:
            mid = 0.5 * (lo + hi)
        cnt = jnp.sum((logits >= mid).astype(jnp.float32), axis=1, keepdims=True)
        ge = cnt >= TOP_K
        lo = jnp.where(ge, mid, lo)
        hi = jnp.where(ge, hi, mid)

    sig = 0.5 + 0.5 * jnp.tanh(0.5 * logits)
    out = jnp.where(logits >= lo, sig * gate, 0.0)
    out_ref[...] = out[None]


@jax.jit
def kernel(context_vec, var_emb, W1, b1, W2, b2, Wg, bg):
    grid = (BATCH, NUM_VARS // ROWS)
    out = pl.pallas_call(
        _fused_kernel,
        grid=grid,
        in_specs=[
            pl.BlockSpec((1, 1, IN_DIM), lambda b, r: (b, 0, 0)),
            pl.BlockSpec((1, IN_DIM, 1), lambda b, r: (b, 0, 0)),
            pl.BlockSpec((IN_DIM, EMBED_DIM), lambda b, r: (0, 0)),
            pl.BlockSpec((1, EMBED_DIM), lambda b, r: (0, 0)),
            pl.BlockSpec((EMBED_DIM, EMBED_DIM), lambda b, r: (0, 0)),
            pl.BlockSpec((1, EMBED_DIM), lambda b, r: (0, 0)),
            pl.BlockSpec((EMBED_DIM, 1), lambda b, r: (0, 0)),
            pl.BlockSpec((1, 1), lambda b, r: (0, 0)),
            pl.BlockSpec((EMBED_DIM, IN_DIM), lambda b, r: (0, 0)),
            pl.BlockSpec((EMBED_DIM, 1), lambda b, r: (0, 0)),
            pl.BlockSpec((EMBED_DIM, EMBED_DIM), lambda b, r: (0, 0)),
            pl.BlockSpec((EMBED_DIM, 1), lambda b, r: (0, 0)),
            pl.BlockSpec((ROWS, EMBED_DIM), lambda b, r: (r, 0)),
            pl.BlockSpec((EMBED_DIM, NUM_VARS), lambda b, r: (0, 0)),
        ],
        out_specs=pl.BlockSpec((1, ROWS, NUM_VARS), lambda b, r: (b, r, 0)),
        out_shape=jax.ShapeDtypeStruct((BATCH, NUM_VARS, NUM_VARS), jnp.float32),
    )(
        context_vec.reshape(BATCH, 1, IN_DIM),
        context_vec.reshape(BATCH, IN_DIM, 1),
        W1.T,
        b1.reshape(1, EMBED_DIM),
        W2.T,
        b2.reshape(1, EMBED_DIM),
        Wg.T,
        bg.reshape(1, 1),
        W1,
        b1.reshape(EMBED_DIM, 1),
        W2,
        b2.reshape(EMBED_DIM, 1),
        var_emb,
        var_emb.T,
    )
    return out


# speedup vs baseline: 1.3024x; 1.0865x over previous
import jax
import jax.numpy as jnp
from jax.experimental import pallas as pl

BATCH = 32
IN_DIM = 512
EMBED_DIM = 32
NUM_VARS = 1024
TOP_K = 32

ROWS = 1024
N_SEARCH = 16


def _fused_kernel(cvr_ref, cvc_ref, w1t_ref, b1r_ref, w2t_ref, b2r_ref,
                  wgt_ref, bg_ref, w1_ref, b1c_ref, w2_ref, b2c_ref,
                  vrows_ref, vt_ref, out_ref):
    h_r = jax.nn.relu(
        jnp.dot(cvr_ref[0], w1t_ref[...], preferred_element_type=jnp.float32)
        + b1r_ref[...])
    c_r = jnp.dot(h_r, w2t_ref[...], preferred_element_type=jnp.float32) + b2r_ref[...]
    gate = jax.nn.sigmoid(
        jnp.dot(c_r, wgt_ref[...], preferred_element_type=jnp.float32)
        + bg_ref[...])[0, 0]

    h_c = jax.nn.relu(
        jnp.dot(w1_ref[...], cvc_ref[0], preferred_element_type=jnp.float32)
        + b1c_ref[...])
    c_c = jnp.dot(w2_ref[...], h_c, preferred_element_type=jnp.float32) + b2c_ref[...]

    sim_rows = vrows_ref[...] * c_r
    sim_t = vt_ref[...] * c_c
    logits = jnp.dot(sim_rows, sim_t, preferred_element_type=jnp.float32)

    mu = jnp.mean(logits, axis=1, keepdims=True)
    m2 = jnp.mean(logits * logits, axis=1, keepdims=True)
    sd = jnp.sqrt(jnp.maximum(m2 - mu * mu, 1e-12))
    lo = mu - 4.0 * sd
    hi = mu + 6.0 * sd
    ge = None
    for it in range(N_SEARCH):
        if it == 0:
            mid = mu + 1.8627 * sd
        elif it == 1:
            mid = jnp.where(ge, mu + 2.35 * sd, mu + 1.45 * sd)
        else:
            mid = 0.5 * (lo + hi)
        cnt = jnp.sum((logits >= mid).astype(jnp.float32), axis=1, keepdims=True)
        ge = cnt >= TOP_K
        lo = jnp.where(ge, mid, lo)
        hi = jnp.where(ge, hi, mid)

    sig = 0.5 + 0.5 * jnp.tanh(0.5 * logits)
    out = jnp.where(logits >= lo, sig * gate, 0.0)
    out_ref[...] = out[None]


@jax.jit
def kernel(context_vec, var_emb, W1, b1, W2, b2, Wg, bg):
    grid = (BATCH, NUM_VARS // ROWS)
    out = pl.pallas_call(
        _fused_kernel,
        grid=grid,
        in_specs=[
            pl.BlockSpec((1, 1, IN_DIM), lambda b, r: (b, 0, 0)),
            pl.BlockSpec((1, IN_DIM, 1), lambda b, r: (b, 0, 0)),
            pl.BlockSpec((IN_DIM, EMBED_DIM), lambda b, r: (0, 0)),
            pl.BlockSpec((1, EMBED_DIM), lambda b, r: (0, 0)),
            pl.BlockSpec((EMBED_DIM, EMBED_DIM), lambda b, r: (0, 0)),
            pl.BlockSpec((1, EMBED_DIM), lambda b, r: (0, 0)),
            pl.BlockSpec((EMBED_DIM, 1), lambda b, r: (0, 0)),
            pl.BlockSpec((1, 1), lambda b, r: (0, 0)),
            pl.BlockSpec((EMBED_DIM, IN_DIM), lambda b, r: (0, 0)),
            pl.BlockSpec((EMBED_DIM, 1), lambda b, r: (0, 0)),
            pl.BlockSpec((EMBED_DIM, EMBED_DIM), lambda b, r: (0, 0)),
            pl.BlockSpec((EMBED_DIM, 1), lambda b, r: (0, 0)),
            pl.BlockSpec((ROWS, EMBED_DIM), lambda b, r: (r, 0)),
            pl.BlockSpec((EMBED_DIM, NUM_VARS), lambda b, r: (0, 0)),
        ],
        out_specs=pl.BlockSpec((1, ROWS, NUM_VARS), lambda b, r: (b, r, 0)),
        out_shape=jax.ShapeDtypeStruct((BATCH, NUM_VARS, NUM_VARS), jnp.float32),
    )(
        context_vec.reshape(BATCH, 1, IN_DIM),
        context_vec.reshape(BATCH, IN_DIM, 1),
        W1.T,
        b1.reshape(1, EMBED_DIM),
        W2.T,
        b2.reshape(1, EMBED_DIM),
        Wg.T,
        bg.reshape(1, 1),
        W1,
        b1.reshape(EMBED_DIM, 1),
        W2,
        b2.reshape(EMBED_DIM, 1),
        var_emb,
        var_emb.T,
    )
    return out
